# Initial kernel scaffold; baseline (speedup 1.0000x reference)
import jax
import jax.numpy as jnp
from jax.experimental import pallas as pl


def kernel(x, mask, edge_type, A, B):
    raise NotImplementedError("write your pallas kernel here")


# speedup vs baseline: 7.5739x; 7.5739x over previous
import functools

import jax
import jax.numpy as jnp
from jax import lax
from jax.experimental import pallas as pl
from jax.experimental.pallas import tpu as pltpu
from jax.experimental.pallas import tpu_sc as plsc

_N = 100000
_DIM = 512
_RANK = 64
_R = 64
_E = 1600000

_INFO = plsc.get_sparse_core_info()
_NC = _INFO.num_cores
_NS = _INFO.num_subcores
_L = _INFO.num_lanes
_NW = _NC * _NS
_EPW = _E // _NW
_VECS = _EPW // _L


@functools.partial(
    pl.kernel,
    mesh=plsc.VectorSubcoreMesh(core_axis_name="c", subcore_axis_name="s"),
    out_type=jax.ShapeDtypeStruct((_NW, _R), jnp.float32),
    scratch_types=[
        pltpu.VMEM((_EPW,), jnp.int32),
        pltpu.VMEM((_L * _R,), jnp.float32),
        pltpu.VMEM((_R,), jnp.float32),
    ],
    compiler_params=pltpu.CompilerParams(needs_layout_passes=False),
)
def _hist_sc(edge_hbm, out_hbm, ids_v, accflat, acc1d):
    c = lax.axis_index("c")
    s = lax.axis_index("s")
    wid = s * _NC + c
    base = wid * _EPW
    pltpu.sync_copy(edge_hbm.at[pl.ds(base, _EPW)], ids_v)

    zero16 = jnp.zeros((_L,), jnp.float32)
    for r in range(_L * _R // _L):
        accflat[pl.ds(r * _L, _L)] = zero16

    lane_off = jnp.arange(_L, dtype=jnp.int32) * _R
    ones = jnp.ones((_L,), jnp.float32)

    def body(i, carry):
        idx = ids_v[pl.ds(i * _L, _L)]
        plsc.addupdate_scatter(accflat, [lane_off + idx], ones)
        return carry

    lax.fori_loop(0, _VECS, body, 0, unroll=8)

    for cc in range(_R // _L):
        a = accflat[pl.ds(cc * _L, _L)]
        for r in range(1, _L):
            a = a + accflat[pl.ds(r * _R + cc * _L, _L)]
        acc1d[pl.ds(cc * _L, _L)] = a
    pltpu.sync_copy(acc1d, out_hbm.at[wid])


@functools.partial(
    pl.pallas_call,
    grid=(1,),
    in_specs=[
        pl.BlockSpec((_NW, _R), lambda i: (0, 0)),
        pl.BlockSpec((_R, _R * _DIM), lambda i: (0, 0)),
    ],
    out_specs=pl.BlockSpec((1, _R * _DIM), lambda i: (0, 0)),
    out_shape=jax.ShapeDtypeStruct((1, _R * _DIM), jnp.float32),
)
def _bavg_tc(pc_ref, b_ref, o_ref):
    counts = jnp.sum(pc_ref[...], axis=0, keepdims=True)
    w = counts / (jnp.sum(counts) + 1e-8)
    o_ref[...] = jnp.dot(w, b_ref[...], preferred_element_type=jnp.float32)


_TM = 1000


@functools.partial(
    pl.pallas_call,
    grid=(_N // _TM,),
    in_specs=[
        pl.BlockSpec((_TM, _DIM), lambda i: (i, 0)),
        pl.BlockSpec((_TM, 1), lambda i: (i, 0)),
        pl.BlockSpec((_DIM, _RANK), lambda i: (0, 0)),
        pl.BlockSpec((_RANK, _DIM), lambda i: (0, 0)),
    ],
    out_specs=pl.BlockSpec((_TM, _DIM), lambda i: (i, 0)),
    out_shape=jax.ShapeDtypeStruct((_N, _DIM), jnp.float32),
    compiler_params=pltpu.CompilerParams(
        dimension_semantics=("arbitrary",)),
)
def _apply_tc(x_ref, m_ref, a_ref, bavg_ref, o_ref):
    x = x_ref[...]
    t = jnp.dot(x * m_ref[...], a_ref[...], preferred_element_type=jnp.float32)
    o_ref[...] = x + jnp.dot(t, bavg_ref[...], preferred_element_type=jnp.float32)


def kernel(x, mask, edge_type, A, B):
    pc = _hist_sc(edge_type)
    bavg = _bavg_tc(pc, B.reshape(_R, _R * _DIM)).reshape(_RANK, _DIM)
    mf = mask.astype(jnp.float32)[:, None]
    return _apply_tc(x, mf, A, bavg)


# speedup vs baseline: 8.7041x; 1.1492x over previous
import functools

import jax
import jax.numpy as jnp
from jax import lax
from jax.experimental import pallas as pl
from jax.experimental.pallas import tpu as pltpu
from jax.experimental.pallas import tpu_sc as plsc

_N = 100000
_DIM = 512
_RANK = 64
_R = 64
_E = 1600000

_INFO = plsc.get_sparse_core_info()
_NC = _INFO.num_cores
_NS = _INFO.num_subcores
_L = _INFO.num_lanes
_NW = _NC * _NS
_EPW = _E // _NW
_VECS = _EPW // _L


@functools.partial(
    pl.kernel,
    mesh=plsc.VectorSubcoreMesh(core_axis_name="c", subcore_axis_name="s"),
    out_type=jax.ShapeDtypeStruct((_NW, _R), jnp.float32),
    scratch_types=[
        pltpu.VMEM((_EPW,), jnp.int32),
        pltpu.VMEM((_L * _R,), jnp.float32),
        pltpu.VMEM((_R,), jnp.float32),
    ],
    compiler_params=pltpu.CompilerParams(needs_layout_passes=False),
)
def _hist_sc(edge_hbm, out_hbm, ids_v, accflat, acc1d):
    c = lax.axis_index("c")
    s = lax.axis_index("s")
    wid = s * _NC + c
    base = wid * _EPW
    pltpu.sync_copy(edge_hbm.at[pl.ds(base, _EPW)], ids_v)

    zero16 = jnp.zeros((_L,), jnp.float32)
    for r in range(_L * _R // _L):
        accflat[pl.ds(r * _L, _L)] = zero16

    lane_off = jnp.arange(_L, dtype=jnp.int32) * _R
    ones = jnp.ones((_L,), jnp.float32)

    def body(i, carry):
        idx = ids_v[pl.ds(i * _L, _L)]
        plsc.addupdate_scatter(accflat, [lane_off + idx], ones)
        return carry

    lax.fori_loop(0, _VECS, body, 0, unroll=8)

    for cc in range(_R // _L):
        a = accflat[pl.ds(cc * _L, _L)]
        for r in range(1, _L):
            a = a + accflat[pl.ds(r * _R + cc * _L, _L)]
        acc1d[pl.ds(cc * _L, _L)] = a
    pltpu.sync_copy(acc1d, out_hbm.at[wid])


@functools.partial(
    pl.pallas_call,
    grid=(1,),
    in_specs=[
        pl.BlockSpec((_NW, _R), lambda i: (0, 0)),
        pl.BlockSpec((_R, _R * _DIM), lambda i: (0, 0)),
    ],
    out_specs=pl.BlockSpec((1, _R * _DIM), lambda i: (0, 0)),
    out_shape=jax.ShapeDtypeStruct((1, _R * _DIM), jnp.float32),
)
def _bavg_tc(pc_ref, b_ref, o_ref):
    counts = jnp.sum(pc_ref[...], axis=0, keepdims=True)
    w = counts / (jnp.sum(counts) + 1e-8)
    o_ref[...] = jnp.dot(w, b_ref[...], preferred_element_type=jnp.float32)


_TM = 2000


@functools.partial(
    pl.pallas_call,
    grid=(_N // _TM,),
    in_specs=[
        pl.BlockSpec((_TM, _DIM), lambda i: (i, 0)),
        pl.BlockSpec((_TM, 1), lambda i: (i, 0)),
        pl.BlockSpec((_DIM, _RANK), lambda i: (0, 0)),
        pl.BlockSpec((_RANK, _DIM), lambda i: (0, 0)),
    ],
    out_specs=pl.BlockSpec((_TM, _DIM), lambda i: (i, 0)),
    out_shape=jax.ShapeDtypeStruct((_N, _DIM), jnp.float32),
    compiler_params=pltpu.CompilerParams(
        dimension_semantics=("arbitrary",)),
)
def _apply_tc(x_ref, m_ref, a_ref, bavg_ref, o_ref):
    x = x_ref[...]
    t = jnp.dot(x * m_ref[...], a_ref[...], preferred_element_type=jnp.float32)
    o_ref[...] = x + jnp.dot(t, bavg_ref[...], preferred_element_type=jnp.float32)


def kernel(x, mask, edge_type, A, B):
    pc = _hist_sc(edge_type)
    bavg = _bavg_tc(pc, B.reshape(_R, _R * _DIM)).reshape(_RANK, _DIM)
    mf = mask.astype(jnp.float32)[:, None]
    return _apply_tc(x, mf, A, bavg)


# speedup vs baseline: 9.1154x; 1.0473x over previous
import functools

import jax
import jax.numpy as jnp
from jax import lax
from jax.experimental import pallas as pl
from jax.experimental.pallas import tpu as pltpu
from jax.experimental.pallas import tpu_sc as plsc

_N = 100000
_DIM = 512
_RANK = 64
_R = 64
_E = 1600000

_INFO = plsc.get_sparse_core_info()
_NC = _INFO.num_cores
_NS = _INFO.num_subcores
_L = _INFO.num_lanes
_NW = _NC * _NS
_EPW = _E // _NW
_VECS = _EPW // _L


@functools.partial(
    pl.kernel,
    mesh=plsc.VectorSubcoreMesh(core_axis_name="c", subcore_axis_name="s"),
    out_type=jax.ShapeDtypeStruct((_NW, _R), jnp.float32),
    scratch_types=[
        pltpu.VMEM((_EPW,), jnp.int32),
        pltpu.VMEM((_L * _R,), jnp.float32),
        pltpu.VMEM((_R,), jnp.float32),
    ],
    compiler_params=pltpu.CompilerParams(needs_layout_passes=False),
)
def _hist_sc(edge_hbm, out_hbm, ids_v, accflat, acc1d):
    c = lax.axis_index("c")
    s = lax.axis_index("s")
    wid = s * _NC + c
    base = wid * _EPW
    pltpu.sync_copy(edge_hbm.at[pl.ds(base, _EPW)], ids_v)

    zero16 = jnp.zeros((_L,), jnp.float32)
    for r in range(_L * _R // _L):
        accflat[pl.ds(r * _L, _L)] = zero16

    lane_off = jnp.arange(_L, dtype=jnp.int32) * _R
    ones = jnp.ones((_L,), jnp.float32)

    def body(i, carry):
        idx = ids_v[pl.ds(i * _L, _L)]
        plsc.addupdate_scatter(accflat, [lane_off + idx], ones)
        return carry

    lax.fori_loop(0, _VECS, body, 0, unroll=8)

    for cc in range(_R // _L):
        a = accflat[pl.ds(cc * _L, _L)]
        for r in range(1, _L):
            a = a + accflat[pl.ds(r * _R + cc * _L, _L)]
        acc1d[pl.ds(cc * _L, _L)] = a
    pltpu.sync_copy(acc1d, out_hbm.at[wid])


@functools.partial(
    pl.pallas_call,
    grid=(1,),
    in_specs=[
        pl.BlockSpec((_NW, _R), lambda i: (0, 0)),
        pl.BlockSpec((_R, _R * _DIM), lambda i: (0, 0)),
    ],
    out_specs=pl.BlockSpec((1, _R * _DIM), lambda i: (0, 0)),
    out_shape=jax.ShapeDtypeStruct((1, _R * _DIM), jnp.float32),
)
def _bavg_tc(pc_ref, b_ref, o_ref):
    counts = jnp.sum(pc_ref[...], axis=0, keepdims=True)
    w = counts / (jnp.sum(counts) + 1e-8)
    o_ref[...] = jnp.dot(w, b_ref[...], preferred_element_type=jnp.float32)


_TM = 4000


@functools.partial(
    pl.pallas_call,
    grid=(_N // _TM,),
    in_specs=[
        pl.BlockSpec((_TM, _DIM), lambda i: (i, 0)),
        pl.BlockSpec((_TM, 1), lambda i: (i, 0)),
        pl.BlockSpec((_DIM, _RANK), lambda i: (0, 0)),
        pl.BlockSpec((_RANK, _DIM), lambda i: (0, 0)),
    ],
    out_specs=pl.BlockSpec((_TM, _DIM), lambda i: (i, 0)),
    out_shape=jax.ShapeDtypeStruct((_N, _DIM), jnp.float32),
    compiler_params=pltpu.CompilerParams(
        dimension_semantics=("arbitrary",)),
)
def _apply_tc(x_ref, m_ref, a_ref, bavg_ref, o_ref):
    x = x_ref[...]
    t = jnp.dot(x * m_ref[...], a_ref[...], preferred_element_type=jnp.float32)
    o_ref[...] = x + jnp.dot(t, bavg_ref[...], preferred_element_type=jnp.float32)


def kernel(x, mask, edge_type, A, B):
    pc = _hist_sc(edge_type)
    bavg = _bavg_tc(pc, B.reshape(_R, _R * _DIM)).reshape(_RANK, _DIM)
    mf = mask.astype(jnp.float32)[:, None]
    return _apply_tc(x, mf, A, bavg)


# speedup vs baseline: 9.1607x; 1.0050x over previous
import functools

import jax
import jax.numpy as jnp
from jax import lax
from jax.experimental import pallas as pl
from jax.experimental.pallas import tpu as pltpu
from jax.experimental.pallas import tpu_sc as plsc

_N = 100000
_DIM = 512
_RANK = 64
_R = 64
_E = 1600000

_INFO = plsc.get_sparse_core_info()
_NC = _INFO.num_cores
_NS = _INFO.num_subcores
_L = _INFO.num_lanes
_NW = _NC * _NS
_EPW = _E // _NW
_VECS = _EPW // _L


@functools.partial(
    pl.kernel,
    mesh=plsc.VectorSubcoreMesh(core_axis_name="c", subcore_axis_name="s"),
    out_type=jax.ShapeDtypeStruct((_NW, _R), jnp.float32),
    scratch_types=[
        pltpu.VMEM((_EPW,), jnp.int32),
        pltpu.VMEM((_L * _R,), jnp.float32),
        pltpu.VMEM((_R,), jnp.float32),
    ],
    compiler_params=pltpu.CompilerParams(needs_layout_passes=False),
)
def _hist_sc(edge_hbm, out_hbm, ids_v, accflat, acc1d):
    c = lax.axis_index("c")
    s = lax.axis_index("s")
    wid = s * _NC + c
    base = wid * _EPW
    pltpu.sync_copy(edge_hbm.at[pl.ds(base, _EPW)], ids_v)

    zero16 = jnp.zeros((_L,), jnp.float32)
    for r in range(_L * _R // _L):
        accflat[pl.ds(r * _L, _L)] = zero16

    lane_off = jnp.arange(_L, dtype=jnp.int32) * _R
    ones = jnp.ones((_L,), jnp.float32)

    def body(i, carry):
        idx = ids_v[pl.ds(i * _L, _L)]
        plsc.addupdate_scatter(accflat, [lane_off + idx], ones)
        return carry

    lax.fori_loop(0, _VECS, body, 0, unroll=8)

    for cc in range(_R // _L):
        a = accflat[pl.ds(cc * _L, _L)]
        for r in range(1, _L):
            a = a + accflat[pl.ds(r * _R + cc * _L, _L)]
        acc1d[pl.ds(cc * _L, _L)] = a
    pltpu.sync_copy(acc1d, out_hbm.at[wid])


@functools.partial(
    pl.pallas_call,
    grid=(1,),
    in_specs=[
        pl.BlockSpec((_NW, _R), lambda i: (0, 0)),
        pl.BlockSpec((_R, _R * _DIM), lambda i: (0, 0)),
    ],
    out_specs=pl.BlockSpec((1, _R * _DIM), lambda i: (0, 0)),
    out_shape=jax.ShapeDtypeStruct((1, _R * _DIM), jnp.float32),
)
def _bavg_tc(pc_ref, b_ref, o_ref):
    counts = jnp.sum(pc_ref[...], axis=0, keepdims=True)
    w = counts / (jnp.sum(counts) + 1e-8)
    o_ref[...] = jnp.dot(w, b_ref[...], preferred_element_type=jnp.float32)


_TM = 5000


@functools.partial(
    pl.pallas_call,
    grid=(_N // _TM,),
    in_specs=[
        pl.BlockSpec((_TM, _DIM), lambda i: (i, 0)),
        pl.BlockSpec((_TM, 1), lambda i: (i, 0)),
        pl.BlockSpec((_DIM, _RANK), lambda i: (0, 0)),
        pl.BlockSpec((_RANK, _DIM), lambda i: (0, 0)),
    ],
    out_specs=pl.BlockSpec((_TM, _DIM), lambda i: (i, 0)),
    out_shape=jax.ShapeDtypeStruct((_N, _DIM), jnp.float32),
    compiler_params=pltpu.CompilerParams(
        dimension_semantics=("arbitrary",)),
)
def _apply_tc(x_ref, m_ref, a_ref, bavg_ref, o_ref):
    x = x_ref[...]
    t = jnp.dot(x * m_ref[...], a_ref[...], preferred_element_type=jnp.float32)
    o_ref[...] = x + jnp.dot(t, bavg_ref[...], preferred_element_type=jnp.float32)


def kernel(x, mask, edge_type, A, B):
    pc = _hist_sc(edge_type)
    bavg = _bavg_tc(pc, B.reshape(_R, _R * _DIM)).reshape(_RANK, _DIM)
    mf = mask.astype(jnp.float32)[:, None]
    return _apply_tc(x, mf, A, bavg)


# speedup vs baseline: 9.2328x; 1.0079x over previous
import functools

import jax
import jax.numpy as jnp
from jax import lax
from jax.experimental import pallas as pl
from jax.experimental.pallas import tpu as pltpu
from jax.experimental.pallas import tpu_sc as plsc

_N = 100000
_DIM = 512
_RANK = 64
_R = 64
_E = 1600000

_INFO = plsc.get_sparse_core_info()
_NC = _INFO.num_cores
_NS = _INFO.num_subcores
_L = _INFO.num_lanes
_NW = _NC * _NS
_EPW = _E // _NW
_NB = 8
_NCH = 5
_CH = _EPW // _NCH
_CVECS = _CH // _L


@functools.partial(
    pl.kernel,
    mesh=plsc.VectorSubcoreMesh(core_axis_name="c", subcore_axis_name="s"),
    out_type=jax.ShapeDtypeStruct((_NW, _R), jnp.float32),
    scratch_types=[
        pltpu.VMEM((_CH,), jnp.int32),
        pltpu.VMEM((_CH,), jnp.int32),
        pltpu.VMEM((_NB * _L * _R,), jnp.float32),
        pltpu.VMEM((_R,), jnp.float32),
        pltpu.SemaphoreType.DMA,
        pltpu.SemaphoreType.DMA,
    ],
    compiler_params=pltpu.CompilerParams(needs_layout_passes=False),
)
def _hist_sc(edge_hbm, out_hbm, ids_a, ids_b, accflat, acc1d, sem0, sem1):
    c = lax.axis_index("c")
    s = lax.axis_index("s")
    wid = s * _NC + c
    base = wid * _EPW
    sems = [sem0, sem1]
    bufs = [ids_a, ids_b]

    cps = [None] * _NCH
    cps[0] = pltpu.async_copy(
        edge_hbm.at[pl.ds(base, _CH)], ids_a, sems[0])

    zero16 = jnp.zeros((_L,), jnp.float32)
    for r in range(_NB * _L * _R // _L):
        accflat[pl.ds(r * _L, _L)] = zero16

    lane_off = jnp.arange(_L, dtype=jnp.int32) * _R
    ones = jnp.ones((_L,), jnp.float32)
    bank_off = [jnp.int32(b * _L * _R) for b in range(_NB)]

    ngroups = _CVECS // _NB

    for ch in range(_NCH):
        cps[ch].wait()
        if ch + 1 < _NCH:
            cps[ch + 1] = pltpu.async_copy(
                edge_hbm.at[pl.ds(base + (ch + 1) * _CH, _CH)],
                bufs[(ch + 1) % 2], sems[(ch + 1) % 2])
        buf = bufs[ch % 2]

        @plsc.parallel_loop(0, ngroups, 1, unroll=2)
        def _(g):
            for b in range(_NB):
                idx = buf[pl.ds((g * _NB + b) * _L, _L)]
                plsc.addupdate_scatter(
                    accflat, [bank_off[b] + lane_off + idx], ones)

        for t in range(ngroups * _NB, _CVECS):
            idx = buf[pl.ds(t * _L, _L)]
            plsc.addupdate_scatter(
                accflat, [bank_off[t % _NB] + lane_off + idx], ones)

    for cc in range(_R // _L):
        a = accflat[pl.ds(cc * _L, _L)]
        first = True
        for b in range(_NB):
            for r in range(_L):
                if first:
                    first = False
                    continue
                a = a + accflat[pl.ds(b * _L * _R + r * _R + cc * _L, _L)]
        acc1d[pl.ds(cc * _L, _L)] = a
    pltpu.sync_copy(acc1d, out_hbm.at[wid])


_TM = 5000


@functools.partial(
    pl.pallas_call,
    grid=(_N // _TM,),
    in_specs=[
        pl.BlockSpec((_NW, _R), lambda i: (0, 0)),
        pl.BlockSpec((_R, _R * _DIM), lambda i: (0, 0)),
        pl.BlockSpec((_TM, _DIM), lambda i: (i, 0)),
        pl.BlockSpec((_TM, 1), lambda i: (i, 0)),
        pl.BlockSpec((_DIM, _RANK), lambda i: (0, 0)),
    ],
    out_specs=pl.BlockSpec((_TM, _DIM), lambda i: (i, 0)),
    out_shape=jax.ShapeDtypeStruct((_N, _DIM), jnp.float32),
    scratch_shapes=[pltpu.VMEM((_RANK, _DIM), jnp.float32)],
    compiler_params=pltpu.CompilerParams(
        dimension_semantics=("arbitrary",)),
)
def _apply_tc(pc_ref, b_ref, x_ref, m_ref, a_ref, o_ref, bavg_s):
    @pl.when(pl.program_id(0) == 0)
    def _():
        counts = jnp.sum(pc_ref[...], axis=0, keepdims=True)
        w = counts / (jnp.sum(counts) + 1e-8)
        val = jnp.dot(w, b_ref[...], preferred_element_type=jnp.float32)
        bavg_s[...] = val.reshape(_RANK, _DIM)

    x = x_ref[...]
    t = jnp.dot(x * m_ref[...], a_ref[...], preferred_element_type=jnp.float32)
    o_ref[...] = x + jnp.dot(t, bavg_s[...], preferred_element_type=jnp.float32)


def kernel(x, mask, edge_type, A, B):
    pc = _hist_sc(edge_type)
    mf = mask.astype(jnp.float32)[:, None]
    return _apply_tc(pc, B.reshape(_R, _R * _DIM), x, mf, A)
